# Initial kernel scaffold; baseline (speedup 1.0000x reference)
import jax
import jax.numpy as jnp
from jax.experimental import pallas as pl


def kernel(x, edge_index, W0l, b0l, W0r, W1l, b1l, W1r, W2l, b2l, W2r):
    raise NotImplementedError("write your pallas kernel here")


# speedup vs baseline: 5.0915x; 5.0915x over previous
import jax
import jax.numpy as jnp
from jax import lax
from jax.experimental import pallas as pl
from jax.experimental.pallas import tpu as pltpu
from jax.experimental.pallas import tpu_sc as plsc

N = 10000
E = 320000
D = 128

NC = 2
NS = 16
NW = NC * NS
EPT = E // NW
CHUNK = 80
NCHUNK = EPT // CHUNK
NP = 10240
RPT = NP // NS
DR = NP // 16
DRT = DR // NS


def _sc_agg_build(compute_deg: bool):
    mesh = plsc.VectorSubcoreMesh(core_axis_name="c", subcore_axis_name="s")
    out_type = [jax.ShapeDtypeStruct((NC * NP, D), jnp.float32)]
    if compute_deg:
        out_type.append(jax.ShapeDtypeStruct((NC * DR, 16), jnp.float32))
    scratch = [
        pltpu.VMEM((CHUNK,), jnp.int32),
        pltpu.VMEM((CHUNK,), jnp.int32),
        pltpu.VMEM((CHUNK, D), jnp.float32),
        pltpu.VMEM_SHARED((NP, D), jnp.float32),
        pltpu.SemaphoreType.DMA,
    ]
    if compute_deg:
        scratch += [
            pltpu.VMEM((DR, 16), jnp.float32),
            pltpu.VMEM((DR,), jnp.int32),
            pltpu.VMEM_SHARED((DR, 16), jnp.float32),
        ]

    def body(*args):
        if compute_deg:
            (y_hbm, src_hbm, dst_hbm, zeros_hbm, zdeg_hbm, iota_hbm,
             agg_out, deg_out,
             src_v, dst_v, rows_v, acc_sh, sem, deg_v, id_v, deg_sh) = args
        else:
            (y_hbm, src_hbm, dst_hbm, zeros_hbm,
             agg_out,
             src_v, dst_v, rows_v, acc_sh, sem) = args
        cid = lax.axis_index("c")
        sid = lax.axis_index("s")
        wid = sid * NC + cid

        pltpu.sync_copy(zeros_hbm, acc_sh.at[pl.ds(sid * RPT, RPT)])
        if compute_deg:
            pltpu.sync_copy(zdeg_hbm, deg_v)
            pltpu.sync_copy(iota_hbm, id_v)
            pltpu.sync_copy(
                zdeg_hbm.at[pl.ds(0, DRT)], deg_sh.at[pl.ds(sid * DRT, DRT)]
            )
        plsc.subcore_barrier()

        ebase = wid * EPT
        ones16 = jnp.full((16,), 1.0, jnp.float32)

        def ebody(c, _):
            off = ebase + c * CHUNK
            pltpu.sync_copy(src_hbm.at[pl.ds(off, CHUNK)], src_v)
            pltpu.sync_copy(dst_hbm.at[pl.ds(off, CHUNK)], dst_v)
            pltpu.async_copy(y_hbm.at[src_v], rows_v, sem).wait()
            pltpu.sync_copy(rows_v, acc_sh.at[dst_v], add=True)
            if compute_deg:
                for j in range(CHUNK // 16):
                    dvals = dst_v[pl.ds(j * 16, 16)]
                    plsc.addupdate_scatter(
                        deg_v, [dvals >> 4, dvals & 15], ones16
                    )
            return 0

        lax.fori_loop(0, NCHUNK, ebody, 0)
        plsc.subcore_barrier()

        pltpu.sync_copy(
            acc_sh.at[pl.ds(sid * RPT, RPT)],
            agg_out.at[pl.ds(cid * NP + sid * RPT, RPT)],
        )
        if compute_deg:
            pltpu.sync_copy(deg_v, deg_sh.at[id_v], add=True)
            plsc.subcore_barrier()
            pltpu.sync_copy(
                deg_sh.at[pl.ds(sid * DRT, DRT)],
                deg_out.at[pl.ds(cid * DR + sid * DRT, DRT)],
            )

    return pl.kernel(
        body,
        out_type=out_type,
        mesh=mesh,
        scratch_types=scratch,
        compiler_params=pltpu.CompilerParams(
            needs_layout_passes=False, use_tc_tiling_on_sc=False
        ),
    )


_sc_agg_deg = _sc_agg_build(True)
_sc_agg = _sc_agg_build(False)


BM = 2000


def _tc_pre_body(x_ref, wl_ref, wr_ref, b_ref, y_ref, z_ref):
    xb = x_ref[...]
    y_ref[...] = jnp.dot(xb, wl_ref[...], preferred_element_type=jnp.float32)
    z_ref[...] = (
        jnp.dot(xb, wr_ref[...], preferred_element_type=jnp.float32) + b_ref[...]
    )


def _tc_pre(x, wlT, wrT, b):
    return pl.pallas_call(
        _tc_pre_body,
        grid=(N // BM,),
        in_specs=[
            pl.BlockSpec((BM, D), lambda i: (i, 0)),
            pl.BlockSpec((D, D), lambda i: (0, 0)),
            pl.BlockSpec((D, D), lambda i: (0, 0)),
            pl.BlockSpec((1, D), lambda i: (0, 0)),
        ],
        out_specs=[
            pl.BlockSpec((BM, D), lambda i: (i, 0)),
            pl.BlockSpec((BM, D), lambda i: (i, 0)),
        ],
        out_shape=[
            jax.ShapeDtypeStruct((N, D), jnp.float32),
            jax.ShapeDtypeStruct((N, D), jnp.float32),
        ],
    )(x, wlT, wrT, b)


def _mean(agg_a_ref, agg_b_ref, deg_a_ref, deg_b_ref, z_ref):
    deg = deg_a_ref[...] + deg_b_ref[...]
    rdeg = 1.0 / jnp.maximum(deg, 1.0)
    return (agg_a_ref[...] + agg_b_ref[...]) * rdeg + z_ref[...]


def _tc_mid_body(agg_a_ref, agg_b_ref, deg_a_ref, deg_b_ref, z_ref,
                 wl_ref, wr_ref, b_ref, y_ref, zo_ref):
    h = jnp.maximum(_mean(agg_a_ref, agg_b_ref, deg_a_ref, deg_b_ref, z_ref), 0.0)
    y_ref[...] = jnp.dot(h, wl_ref[...], preferred_element_type=jnp.float32)
    zo_ref[...] = (
        jnp.dot(h, wr_ref[...], preferred_element_type=jnp.float32) + b_ref[...]
    )


def _tc_mid(agg_a, agg_b, deg_a, deg_b, z, wlT, wrT, b):
    return pl.pallas_call(
        _tc_mid_body,
        grid=(N // BM,),
        in_specs=[
            pl.BlockSpec((BM, D), lambda i: (i, 0)),
            pl.BlockSpec((BM, D), lambda i: (i, 0)),
            pl.BlockSpec((BM, 1), lambda i: (i, 0)),
            pl.BlockSpec((BM, 1), lambda i: (i, 0)),
            pl.BlockSpec((BM, D), lambda i: (i, 0)),
            pl.BlockSpec((D, D), lambda i: (0, 0)),
            pl.BlockSpec((D, D), lambda i: (0, 0)),
            pl.BlockSpec((1, D), lambda i: (0, 0)),
        ],
        out_specs=[
            pl.BlockSpec((BM, D), lambda i: (i, 0)),
            pl.BlockSpec((BM, D), lambda i: (i, 0)),
        ],
        out_shape=[
            jax.ShapeDtypeStruct((N, D), jnp.float32),
            jax.ShapeDtypeStruct((N, D), jnp.float32),
        ],
    )(agg_a, agg_b, deg_a, deg_b, z, wlT, wrT, b)


def _tc_fin_body(agg_a_ref, agg_b_ref, deg_a_ref, deg_b_ref, z_ref, o_ref):
    o_ref[...] = _mean(agg_a_ref, agg_b_ref, deg_a_ref, deg_b_ref, z_ref)


def _tc_fin(agg_a, agg_b, deg_a, deg_b, z):
    return pl.pallas_call(
        _tc_fin_body,
        grid=(N // BM,),
        in_specs=[
            pl.BlockSpec((BM, D), lambda i: (i, 0)),
            pl.BlockSpec((BM, D), lambda i: (i, 0)),
            pl.BlockSpec((BM, 1), lambda i: (i, 0)),
            pl.BlockSpec((BM, 1), lambda i: (i, 0)),
            pl.BlockSpec((BM, D), lambda i: (i, 0)),
        ],
        out_specs=pl.BlockSpec((BM, D), lambda i: (i, 0)),
        out_shape=jax.ShapeDtypeStruct((N, D), jnp.float32),
    )(agg_a, agg_b, deg_a, deg_b, z)


def kernel(x, edge_index, W0l, b0l, W0r, W1l, b1l, W1r, W2l, b2l, W2r):
    src = edge_index[0].astype(jnp.int32)
    dst = edge_index[1].astype(jnp.int32)
    zeros = jnp.zeros((RPT, D), jnp.float32)
    zdeg = jnp.zeros((DR, 16), jnp.float32)
    iota = jnp.arange(DR, dtype=jnp.int32)

    y0, z0 = _tc_pre(x, W0l.T, W0r.T, b0l.reshape(1, D))
    agg0, deg_parts = _sc_agg_deg(y0, src, dst, zeros, zdeg, iota)
    dp = deg_parts.reshape(NC, NP)
    deg_a = dp[0, :N].reshape(N, 1)
    deg_b = dp[1, :N].reshape(N, 1)
    a0a, a0b = agg0[:N], agg0[NP:NP + N]

    y1, z1 = _tc_mid(a0a, a0b, deg_a, deg_b, z0, W1l.T, W1r.T, b1l.reshape(1, D))
    (agg1,) = _sc_agg(y1, src, dst, zeros)
    a1a, a1b = agg1[:N], agg1[NP:NP + N]

    y2, z2 = _tc_mid(a1a, a1b, deg_a, deg_b, z1, W2l.T, W2r.T, b2l.reshape(1, D))
    (agg2,) = _sc_agg(y2, src, dst, zeros)
    a2a, a2b = agg2[:N], agg2[NP:NP + N]

    return _tc_fin(a2a, a2b, deg_a, deg_b, z2)


# speedup vs baseline: 11.2488x; 2.2093x over previous
import jax
import jax.numpy as jnp
from jax import lax
from jax.experimental import pallas as pl
from jax.experimental.pallas import tpu as pltpu
from jax.experimental.pallas import tpu_sc as plsc

N = 10000
E = 320000
D = 128

NC = 2
NS = 16
NW = NC * NS
EPT = E // NW
CHUNK = 80
NCHUNK = EPT // CHUNK
ECH = E // CHUNK
NP = 10240
RPT = NP // NS
DR = NP // 16
DRT = DR // NS


def _sc_agg_make():
    mesh = plsc.VectorSubcoreMesh(core_axis_name="c", subcore_axis_name="s")
    scratch = (
        [pltpu.VMEM((CHUNK, D), jnp.float32) for _ in range(4)]
        + [pltpu.VMEM((2, CHUNK), jnp.int32) for _ in range(8)]
        + [pltpu.VMEM_SHARED((NP, D), jnp.float32)]
        + [pltpu.SemaphoreType.DMA for _ in range(16)]
    )

    def body(y_hbm, eidx_hbm, zeros_hbm, agg_out, *rest):
        rb = rest[0:4]
        ib = rest[4:12]
        acc_sh = rest[12]
        si = rest[13:21]
        sg = rest[21:25]
        ss = rest[25:29]
        cid = lax.axis_index("c")
        sid = lax.axis_index("s")
        wid = sid * NC + cid

        pltpu.sync_copy(zeros_hbm, acc_sh.at[pl.ds(sid * RPT, RPT)])
        plsc.subcore_barrier()

        cbase = wid * NCHUNK

        def idesc(c, j):
            return pltpu.make_async_copy(
                eidx_hbm.at[cbase + c], ib[j % 8], si[j % 8]
            )

        def gdesc(c, j):
            return pltpu.make_async_copy(
                y_hbm.at[ib[j % 8].at[0]], rb[j % 4], sg[j % 4]
            )

        def sdesc(c, j):
            return pltpu.make_async_copy(
                rb[j % 4], acc_sh.at[ib[j % 8].at[1]], ss[j % 4]
            )

        for j in range(4):
            idesc(j, j).start()

        def step(c, j):
            @pl.when(jnp.logical_and(c >= 4, c <= NCHUNK + 3))
            def _():
                sdesc(c - 4, j - 4).wait()

            @pl.when(c + 4 <= NCHUNK - 1)
            def _():
                idesc(c + 4, j + 4).start()

            @pl.when(c <= NCHUNK - 1)
            def _():
                idesc(c, j).wait()
                gdesc(c, j).start()

            @pl.when(jnp.logical_and(c >= 1, c <= NCHUNK))
            def _():
                gdesc(c - 1, j - 1).wait()
                sdesc(c - 1, j - 1).start(add=True)

        def kbody(k, _):
            c0 = 8 * k
            for j in range(8):
                step(c0 + j, j)
            return 0

        lax.fori_loop(0, (NCHUNK + 11) // 8, kbody, 0)
        plsc.subcore_barrier()

        pltpu.sync_copy(
            acc_sh.at[pl.ds(sid * RPT, RPT)],
            agg_out.at[pl.ds(cid * NP + sid * RPT, RPT)],
        )

    return pl.kernel(
        body,
        out_type=jax.ShapeDtypeStruct((NC * NP, D), jnp.float32),
        mesh=mesh,
        scratch_types=scratch,
        compiler_params=pltpu.CompilerParams(
            needs_layout_passes=False, use_tc_tiling_on_sc=False
        ),
    )


def _sc_deg_make():
    mesh = plsc.VectorSubcoreMesh(core_axis_name="c", subcore_axis_name="s")
    scratch = [
        pltpu.VMEM((EPT,), jnp.int32),
        pltpu.VMEM((DR, 16), jnp.float32),
        pltpu.VMEM((DR,), jnp.int32),
        pltpu.VMEM_SHARED((DR, 16), jnp.float32),
    ]

    def body(dst_hbm, zdeg_hbm, iota_hbm, deg_out, dst_v, deg_v, id_v, deg_sh):
        cid = lax.axis_index("c")
        sid = lax.axis_index("s")
        wid = sid * NC + cid

        pltpu.sync_copy(dst_hbm.at[pl.ds(wid * EPT, EPT)], dst_v)
        pltpu.sync_copy(zdeg_hbm, deg_v)
        pltpu.sync_copy(iota_hbm, id_v)
        pltpu.sync_copy(
            zdeg_hbm.at[pl.ds(0, DRT)], deg_sh.at[pl.ds(sid * DRT, DRT)]
        )
        plsc.subcore_barrier()

        ones16 = jnp.full((16,), 1.0, jnp.float32)

        def dbody(i, _):
            dvals = dst_v[pl.ds(i * 16, 16)]
            plsc.addupdate_scatter(deg_v, [dvals >> 4, dvals & 15], ones16)
            return 0

        lax.fori_loop(0, EPT // 16, dbody, 0)

        pltpu.sync_copy(deg_v, deg_sh.at[id_v], add=True)
        plsc.subcore_barrier()
        pltpu.sync_copy(
            deg_sh.at[pl.ds(sid * DRT, DRT)],
            deg_out.at[pl.ds(cid * DR + sid * DRT, DRT)],
        )

    return pl.kernel(
        body,
        out_type=jax.ShapeDtypeStruct((NC * DR, 16), jnp.float32),
        mesh=mesh,
        scratch_types=scratch,
        compiler_params=pltpu.CompilerParams(
            needs_layout_passes=False, use_tc_tiling_on_sc=False
        ),
    )


_sc_agg = _sc_agg_make()
_sc_deg = _sc_deg_make()


BM = 2000


def _tc_pre_body(x_ref, wl_ref, wr_ref, b_ref, y_ref, z_ref):
    xb = x_ref[...]
    y_ref[...] = jnp.dot(xb, wl_ref[...], preferred_element_type=jnp.float32)
    z_ref[...] = (
        jnp.dot(xb, wr_ref[...], preferred_element_type=jnp.float32) + b_ref[...]
    )


def _tc_pre(x, wlT, wrT, b):
    return pl.pallas_call(
        _tc_pre_body,
        grid=(N // BM,),
        in_specs=[
            pl.BlockSpec((BM, D), lambda i: (i, 0)),
            pl.BlockSpec((D, D), lambda i: (0, 0)),
            pl.BlockSpec((D, D), lambda i: (0, 0)),
            pl.BlockSpec((1, D), lambda i: (0, 0)),
        ],
        out_specs=[
            pl.BlockSpec((BM, D), lambda i: (i, 0)),
            pl.BlockSpec((BM, D), lambda i: (i, 0)),
        ],
        out_shape=[
            jax.ShapeDtypeStruct((N, D), jnp.float32),
            jax.ShapeDtypeStruct((N, D), jnp.float32),
        ],
    )(x, wlT, wrT, b)


def _mean(agg_a_ref, agg_b_ref, deg_a_ref, deg_b_ref, z_ref):
    deg = deg_a_ref[...] + deg_b_ref[...]
    rdeg = 1.0 / jnp.maximum(deg, 1.0)
    return (agg_a_ref[...] + agg_b_ref[...]) * rdeg + z_ref[...]


def _tc_mid_body(agg_a_ref, agg_b_ref, deg_a_ref, deg_b_ref, z_ref,
                 wl_ref, wr_ref, b_ref, y_ref, zo_ref):
    h = jnp.maximum(_mean(agg_a_ref, agg_b_ref, deg_a_ref, deg_b_ref, z_ref), 0.0)
    y_ref[...] = jnp.dot(h, wl_ref[...], preferred_element_type=jnp.float32)
    zo_ref[...] = (
        jnp.dot(h, wr_ref[...], preferred_element_type=jnp.float32) + b_ref[...]
    )


def _tc_mid(agg_a, agg_b, deg_a, deg_b, z, wlT, wrT, b):
    return pl.pallas_call(
        _tc_mid_body,
        grid=(N // BM,),
        in_specs=[
            pl.BlockSpec((BM, D), lambda i: (i, 0)),
            pl.BlockSpec((BM, D), lambda i: (i, 0)),
            pl.BlockSpec((BM, 1), lambda i: (i, 0)),
            pl.BlockSpec((BM, 1), lambda i: (i, 0)),
            pl.BlockSpec((BM, D), lambda i: (i, 0)),
            pl.BlockSpec((D, D), lambda i: (0, 0)),
            pl.BlockSpec((D, D), lambda i: (0, 0)),
            pl.BlockSpec((1, D), lambda i: (0, 0)),
        ],
        out_specs=[
            pl.BlockSpec((BM, D), lambda i: (i, 0)),
            pl.BlockSpec((BM, D), lambda i: (i, 0)),
        ],
        out_shape=[
            jax.ShapeDtypeStruct((N, D), jnp.float32),
            jax.ShapeDtypeStruct((N, D), jnp.float32),
        ],
    )(agg_a, agg_b, deg_a, deg_b, z, wlT, wrT, b)


def _tc_fin_body(agg_a_ref, agg_b_ref, deg_a_ref, deg_b_ref, z_ref, o_ref):
    o_ref[...] = _mean(agg_a_ref, agg_b_ref, deg_a_ref, deg_b_ref, z_ref)


def _tc_fin(agg_a, agg_b, deg_a, deg_b, z):
    return pl.pallas_call(
        _tc_fin_body,
        grid=(N // BM,),
        in_specs=[
            pl.BlockSpec((BM, D), lambda i: (i, 0)),
            pl.BlockSpec((BM, D), lambda i: (i, 0)),
            pl.BlockSpec((BM, 1), lambda i: (i, 0)),
            pl.BlockSpec((BM, 1), lambda i: (i, 0)),
            pl.BlockSpec((BM, D), lambda i: (i, 0)),
        ],
        out_specs=pl.BlockSpec((BM, D), lambda i: (i, 0)),
        out_shape=jax.ShapeDtypeStruct((N, D), jnp.float32),
    )(agg_a, agg_b, deg_a, deg_b, z)


def kernel(x, edge_index, W0l, b0l, W0r, W1l, b1l, W1r, W2l, b2l, W2r):
    src = edge_index[0].astype(jnp.int32)
    dst = edge_index[1].astype(jnp.int32)
    eidx = jnp.stack([src.reshape(ECH, CHUNK), dst.reshape(ECH, CHUNK)], axis=1)
    zeros = jnp.zeros((RPT, D), jnp.float32)
    zdeg = jnp.zeros((DR, 16), jnp.float32)
    iota = jnp.arange(DR, dtype=jnp.int32)

    deg_parts = _sc_deg(dst, zdeg, iota)
    dp = deg_parts.reshape(NC, NP)
    deg_a = dp[0, :N].reshape(N, 1)
    deg_b = dp[1, :N].reshape(N, 1)

    y0, z0 = _tc_pre(x, W0l.T, W0r.T, b0l.reshape(1, D))
    agg0 = _sc_agg(y0, eidx, zeros)
    a0a, a0b = agg0[:N], agg0[NP:NP + N]

    y1, z1 = _tc_mid(a0a, a0b, deg_a, deg_b, z0, W1l.T, W1r.T, b1l.reshape(1, D))
    agg1 = _sc_agg(y1, eidx, zeros)
    a1a, a1b = agg1[:N], agg1[NP:NP + N]

    y2, z2 = _tc_mid(a1a, a1b, deg_a, deg_b, z1, W2l.T, W2r.T, b2l.reshape(1, D))
    agg2 = _sc_agg(y2, eidx, zeros)
    a2a, a2b = agg2[:N], agg2[NP:NP + N]

    return _tc_fin(a2a, a2b, deg_a, deg_b, z2)


# speedup vs baseline: 13.8023x; 1.2270x over previous
import jax
import jax.numpy as jnp
from jax import lax
from jax.experimental import pallas as pl
from jax.experimental.pallas import tpu as pltpu
from jax.experimental.pallas import tpu_sc as plsc

N = 10000
E = 320000
D = 128

NC = 2
NS = 16
NW = NC * NS
EPT = E // NW
CHUNK = 125
NCHUNK = EPT // CHUNK
ECH = E // CHUNK
NP = 10240
RPT = NP // NS
DR = NP // 16
DRT = DR // NS


def _sc_agg_make():
    mesh = plsc.VectorSubcoreMesh(core_axis_name="c", subcore_axis_name="s")
    scratch = (
        [pltpu.VMEM((CHUNK, D), jnp.bfloat16) for _ in range(8)]
        + [pltpu.VMEM((NCHUNK, 2, CHUNK), jnp.int32)]
        + [pltpu.VMEM_SHARED((NP, D), jnp.bfloat16)]
        + [pltpu.SemaphoreType.DMA for _ in range(16)]
    )

    def body(y_hbm, eidx_hbm, zeros_hbm, agg_out, *rest):
        rb = rest[0:8]
        eidx_v = rest[8]
        acc_sh = rest[9]
        sg = rest[10:18]
        ss = rest[18:26]
        cid = lax.axis_index("c")
        sid = lax.axis_index("s")
        wid = sid * NC + cid

        pltpu.sync_copy(eidx_hbm.at[pl.ds(wid * NCHUNK, NCHUNK)], eidx_v)
        pltpu.sync_copy(zeros_hbm, acc_sh.at[pl.ds(sid * RPT, RPT)])
        plsc.subcore_barrier()

        def gdesc(c, j):
            return pltpu.make_async_copy(
                y_hbm.at[eidx_v.at[c, 0]], rb[j % 8], sg[j % 8]
            )

        def sdesc(c, j):
            return pltpu.make_async_copy(
                rb[j % 8], acc_sh.at[eidx_v.at[c, 1]], ss[j % 8]
            )

        def step(c, j):
            @pl.when(jnp.logical_and(c >= 8, c <= NCHUNK + 7))
            def _():
                sdesc(c - 8, j).wait()

            @pl.when(c <= NCHUNK - 1)
            def _():
                gdesc(c, j).start()

            @pl.when(jnp.logical_and(c >= 1, c <= NCHUNK))
            def _():
                gdesc(c - 1, j - 1).wait()
                sdesc(c - 1, j - 1).start(add=True)

        def kbody(k, _):
            c0 = 8 * k
            for j in range(8):
                step(c0 + j, j)
            return 0

        lax.fori_loop(0, (NCHUNK + 15) // 8, kbody, 0)
        plsc.subcore_barrier()

        pltpu.sync_copy(
            acc_sh.at[pl.ds(sid * RPT, RPT)],
            agg_out.at[pl.ds(cid * NP + sid * RPT, RPT)],
        )

    return pl.kernel(
        body,
        out_type=jax.ShapeDtypeStruct((NC * NP, D), jnp.bfloat16),
        mesh=mesh,
        scratch_types=scratch,
        compiler_params=pltpu.CompilerParams(
            needs_layout_passes=False, use_tc_tiling_on_sc=False
        ),
    )


def _sc_deg_make():
    mesh = plsc.VectorSubcoreMesh(core_axis_name="c", subcore_axis_name="s")
    scratch = [
        pltpu.VMEM((EPT,), jnp.int32),
        pltpu.VMEM((DR, 16), jnp.float32),
        pltpu.VMEM((DR,), jnp.int32),
        pltpu.VMEM_SHARED((DR, 16), jnp.float32),
    ]

    def body(dst_hbm, zdeg_hbm, iota_hbm, deg_out, dst_v, deg_v, id_v, deg_sh):
        cid = lax.axis_index("c")
        sid = lax.axis_index("s")
        wid = sid * NC + cid

        pltpu.sync_copy(dst_hbm.at[pl.ds(wid * EPT, EPT)], dst_v)
        pltpu.sync_copy(zdeg_hbm, deg_v)
        pltpu.sync_copy(iota_hbm, id_v)
        pltpu.sync_copy(
            zdeg_hbm.at[pl.ds(0, DRT)], deg_sh.at[pl.ds(sid * DRT, DRT)]
        )
        plsc.subcore_barrier()

        ones16 = jnp.full((16,), 1.0, jnp.float32)

        def dbody(i, _):
            dvals = dst_v[pl.ds(i * 16, 16)]
            plsc.addupdate_scatter(deg_v, [dvals >> 4, dvals & 15], ones16)
            return 0

        lax.fori_loop(0, EPT // 16, dbody, 0)

        pltpu.sync_copy(deg_v, deg_sh.at[id_v], add=True)
        plsc.subcore_barrier()
        pltpu.sync_copy(
            deg_sh.at[pl.ds(sid * DRT, DRT)],
            deg_out.at[pl.ds(cid * DR + sid * DRT, DRT)],
        )

    return pl.kernel(
        body,
        out_type=jax.ShapeDtypeStruct((NC * DR, 16), jnp.float32),
        mesh=mesh,
        scratch_types=scratch,
        compiler_params=pltpu.CompilerParams(
            needs_layout_passes=False, use_tc_tiling_on_sc=False
        ),
    )


_sc_agg = _sc_agg_make()
_sc_deg = _sc_deg_make()


BM = 2000


def _tc_pre_body(x_ref, wl_ref, wr_ref, b_ref, y_ref, z_ref):
    xb = x_ref[...]
    y_ref[...] = jnp.dot(
        xb, wl_ref[...], preferred_element_type=jnp.float32
    ).astype(jnp.bfloat16)
    z_ref[...] = (
        jnp.dot(xb, wr_ref[...], preferred_element_type=jnp.float32) + b_ref[...]
    )


def _tc_pre(x, wlT, wrT, b):
    return pl.pallas_call(
        _tc_pre_body,
        grid=(N // BM,),
        in_specs=[
            pl.BlockSpec((BM, D), lambda i: (i, 0)),
            pl.BlockSpec((D, D), lambda i: (0, 0)),
            pl.BlockSpec((D, D), lambda i: (0, 0)),
            pl.BlockSpec((1, D), lambda i: (0, 0)),
        ],
        out_specs=[
            pl.BlockSpec((BM, D), lambda i: (i, 0)),
            pl.BlockSpec((BM, D), lambda i: (i, 0)),
        ],
        out_shape=[
            jax.ShapeDtypeStruct((N, D), jnp.bfloat16),
            jax.ShapeDtypeStruct((N, D), jnp.float32),
        ],
    )(x, wlT, wrT, b)


def _mean(agg_a_ref, agg_b_ref, deg_a_ref, deg_b_ref, z_ref):
    deg = deg_a_ref[...] + deg_b_ref[...]
    rdeg = 1.0 / jnp.maximum(deg, 1.0)
    agg = agg_a_ref[...].astype(jnp.float32) + agg_b_ref[...].astype(jnp.float32)
    return agg * rdeg + z_ref[...]


def _tc_mid_body(agg_a_ref, agg_b_ref, deg_a_ref, deg_b_ref, z_ref,
                 wl_ref, wr_ref, b_ref, y_ref, zo_ref):
    h = jnp.maximum(_mean(agg_a_ref, agg_b_ref, deg_a_ref, deg_b_ref, z_ref), 0.0)
    y_ref[...] = jnp.dot(
        h, wl_ref[...], preferred_element_type=jnp.float32
    ).astype(jnp.bfloat16)
    zo_ref[...] = (
        jnp.dot(h, wr_ref[...], preferred_element_type=jnp.float32) + b_ref[...]
    )


def _tc_mid(agg_a, agg_b, deg_a, deg_b, z, wlT, wrT, b):
    return pl.pallas_call(
        _tc_mid_body,
        grid=(N // BM,),
        in_specs=[
            pl.BlockSpec((BM, D), lambda i: (i, 0)),
            pl.BlockSpec((BM, D), lambda i: (i, 0)),
            pl.BlockSpec((BM, 1), lambda i: (i, 0)),
            pl.BlockSpec((BM, 1), lambda i: (i, 0)),
            pl.BlockSpec((BM, D), lambda i: (i, 0)),
            pl.BlockSpec((D, D), lambda i: (0, 0)),
            pl.BlockSpec((D, D), lambda i: (0, 0)),
            pl.BlockSpec((1, D), lambda i: (0, 0)),
        ],
        out_specs=[
            pl.BlockSpec((BM, D), lambda i: (i, 0)),
            pl.BlockSpec((BM, D), lambda i: (i, 0)),
        ],
        out_shape=[
            jax.ShapeDtypeStruct((N, D), jnp.bfloat16),
            jax.ShapeDtypeStruct((N, D), jnp.float32),
        ],
    )(agg_a, agg_b, deg_a, deg_b, z, wlT, wrT, b)


def _tc_fin_body(agg_a_ref, agg_b_ref, deg_a_ref, deg_b_ref, z_ref, o_ref):
    o_ref[...] = _mean(agg_a_ref, agg_b_ref, deg_a_ref, deg_b_ref, z_ref)


def _tc_fin(agg_a, agg_b, deg_a, deg_b, z):
    return pl.pallas_call(
        _tc_fin_body,
        grid=(N // BM,),
        in_specs=[
            pl.BlockSpec((BM, D), lambda i: (i, 0)),
            pl.BlockSpec((BM, D), lambda i: (i, 0)),
            pl.BlockSpec((BM, 1), lambda i: (i, 0)),
            pl.BlockSpec((BM, 1), lambda i: (i, 0)),
            pl.BlockSpec((BM, D), lambda i: (i, 0)),
        ],
        out_specs=pl.BlockSpec((BM, D), lambda i: (i, 0)),
        out_shape=jax.ShapeDtypeStruct((N, D), jnp.float32),
    )(agg_a, agg_b, deg_a, deg_b, z)


def kernel(x, edge_index, W0l, b0l, W0r, W1l, b1l, W1r, W2l, b2l, W2r):
    src = edge_index[0].astype(jnp.int32)
    dst = edge_index[1].astype(jnp.int32)
    eidx = jnp.stack([src.reshape(ECH, CHUNK), dst.reshape(ECH, CHUNK)], axis=1)
    zeros = jnp.zeros((RPT, D), jnp.bfloat16)
    zdeg = jnp.zeros((DR, 16), jnp.float32)
    iota = jnp.arange(DR, dtype=jnp.int32)

    deg_parts = _sc_deg(dst, zdeg, iota)
    dp = deg_parts.reshape(NC, NP)
    deg_a = dp[0, :N].reshape(N, 1)
    deg_b = dp[1, :N].reshape(N, 1)

    y0, z0 = _tc_pre(x, W0l.T, W0r.T, b0l.reshape(1, D))
    agg0 = _sc_agg(y0, eidx, zeros)
    a0a, a0b = agg0[:N], agg0[NP:NP + N]

    y1, z1 = _tc_mid(a0a, a0b, deg_a, deg_b, z0, W1l.T, W1r.T, b1l.reshape(1, D))
    agg1 = _sc_agg(y1, eidx, zeros)
    a1a, a1b = agg1[:N], agg1[NP:NP + N]

    y2, z2 = _tc_mid(a1a, a1b, deg_a, deg_b, z1, W2l.T, W2r.T, b2l.reshape(1, D))
    agg2 = _sc_agg(y2, eidx, zeros)
    a2a, a2b = agg2[:N], agg2[NP:NP + N]

    return _tc_fin(a2a, a2b, deg_a, deg_b, z2)


# speedup vs baseline: 14.5034x; 1.0508x over previous
import jax
import jax.numpy as jnp
from jax import lax
from jax.experimental import pallas as pl
from jax.experimental.pallas import tpu as pltpu
from jax.experimental.pallas import tpu_sc as plsc

N = 10000
E = 320000
D = 128

NC = 2
NS = 16
NW = NC * NS
EPT = E // NW
CHUNK = 125
NCHUNK = EPT // CHUNK
ECH = E // CHUNK
NP = 10240
RPT = NP // NS
DR = NP // 16
DRT = DR // NS


def _sc_agg_make():
    mesh = plsc.VectorSubcoreMesh(core_axis_name="c", subcore_axis_name="s")
    scratch = (
        [pltpu.VMEM((CHUNK, D), jnp.bfloat16) for _ in range(8)]
        + [pltpu.VMEM((NCHUNK, 2, CHUNK), jnp.int32)]
        + [pltpu.VMEM_SHARED((NP, D), jnp.bfloat16)]
        + [pltpu.SemaphoreType.DMA for _ in range(16)]
    )

    def body(y_hbm, eidx_hbm, zeros_hbm, agg_a_out, agg_b_out, *rest):
        rb = rest[0:8]
        eidx_v = rest[8]
        acc_sh = rest[9]
        sg = rest[10:18]
        ss = rest[18:26]
        cid = lax.axis_index("c")
        sid = lax.axis_index("s")
        wid = sid * NC + cid

        pltpu.sync_copy(eidx_hbm.at[pl.ds(wid * NCHUNK, NCHUNK)], eidx_v)
        pltpu.sync_copy(zeros_hbm, acc_sh.at[pl.ds(sid * RPT, RPT)])
        plsc.subcore_barrier()

        def gdesc(c, j):
            return pltpu.make_async_copy(
                y_hbm.at[eidx_v.at[c, 0]], rb[j % 8], sg[j % 8]
            )

        def sdesc(c, j):
            return pltpu.make_async_copy(
                rb[j % 8], acc_sh.at[eidx_v.at[c, 1]], ss[j % 8]
            )

        def step(c, j):
            @pl.when(jnp.logical_and(c >= 8, c <= NCHUNK + 7))
            def _():
                sdesc(c - 8, j).wait()

            @pl.when(c <= NCHUNK - 1)
            def _():
                gdesc(c, j).start()

            @pl.when(jnp.logical_and(c >= 1, c <= NCHUNK))
            def _():
                gdesc(c - 1, j - 1).wait()
                sdesc(c - 1, j - 1).start(add=True)

        def kbody(k, _):
            c0 = 8 * k
            for j in range(8):
                step(c0 + j, j)
            return 0

        lax.fori_loop(0, (NCHUNK + 15) // 8, kbody, 0)
        plsc.subcore_barrier()

        @pl.when(cid == 0)
        def _():
            pltpu.sync_copy(
                acc_sh.at[pl.ds(sid * RPT, RPT)],
                agg_a_out.at[pl.ds(sid * RPT, RPT)],
            )

        @pl.when(cid == 1)
        def _():
            pltpu.sync_copy(
                acc_sh.at[pl.ds(sid * RPT, RPT)],
                agg_b_out.at[pl.ds(sid * RPT, RPT)],
            )

    return pl.kernel(
        body,
        out_type=[
            jax.ShapeDtypeStruct((NP, D), jnp.bfloat16),
            jax.ShapeDtypeStruct((NP, D), jnp.bfloat16),
        ],
        mesh=mesh,
        scratch_types=scratch,
        compiler_params=pltpu.CompilerParams(
            needs_layout_passes=False, use_tc_tiling_on_sc=False
        ),
    )


def _sc_deg_make():
    mesh = plsc.VectorSubcoreMesh(core_axis_name="c", subcore_axis_name="s")
    scratch = [
        pltpu.VMEM((EPT,), jnp.int32),
        pltpu.VMEM((DR, 16), jnp.float32),
        pltpu.VMEM((DR,), jnp.int32),
        pltpu.VMEM_SHARED((DR, 16), jnp.float32),
    ]

    def body(dst_hbm, zdeg_hbm, iota_hbm, deg_out, dst_v, deg_v, id_v, deg_sh):
        cid = lax.axis_index("c")
        sid = lax.axis_index("s")
        wid = sid * NC + cid

        pltpu.sync_copy(dst_hbm.at[pl.ds(wid * EPT, EPT)], dst_v)
        pltpu.sync_copy(zdeg_hbm, deg_v)
        pltpu.sync_copy(iota_hbm, id_v)
        pltpu.sync_copy(
            zdeg_hbm.at[pl.ds(0, DRT)], deg_sh.at[pl.ds(sid * DRT, DRT)]
        )
        plsc.subcore_barrier()

        ones16 = jnp.full((16,), 1.0, jnp.float32)

        def dbody(i, _):
            dvals = dst_v[pl.ds(i * 16, 16)]
            plsc.addupdate_scatter(deg_v, [dvals >> 4, dvals & 15], ones16)
            return 0

        lax.fori_loop(0, EPT // 16, dbody, 0)

        pltpu.sync_copy(deg_v, deg_sh.at[id_v], add=True)
        plsc.subcore_barrier()
        pltpu.sync_copy(
            deg_sh.at[pl.ds(sid * DRT, DRT)],
            deg_out.at[pl.ds(cid * DR + sid * DRT, DRT)],
        )

    return pl.kernel(
        body,
        out_type=jax.ShapeDtypeStruct((NC * DR, 16), jnp.float32),
        mesh=mesh,
        scratch_types=scratch,
        compiler_params=pltpu.CompilerParams(
            needs_layout_passes=False, use_tc_tiling_on_sc=False
        ),
    )


_sc_agg = _sc_agg_make()
_sc_deg = _sc_deg_make()


BM = 2000


def _tc_pre_body(x_ref, wl_ref, wr_ref, b_ref, y_ref, z_ref):
    xb = x_ref[...]
    y = jnp.dot(xb, wl_ref[...], preferred_element_type=jnp.float32)
    y_ref[...] = y.astype(jnp.bfloat16).reshape(BM * D)
    z_ref[...] = (
        jnp.dot(xb, wr_ref[...], preferred_element_type=jnp.float32) + b_ref[...]
    )


def _tc_pre(x, wlT, wrT, b):
    return pl.pallas_call(
        _tc_pre_body,
        grid=(N // BM,),
        in_specs=[
            pl.BlockSpec((BM, D), lambda i: (i, 0)),
            pl.BlockSpec((D, D), lambda i: (0, 0)),
            pl.BlockSpec((D, D), lambda i: (0, 0)),
            pl.BlockSpec((1, D), lambda i: (0, 0)),
        ],
        out_specs=[
            pl.BlockSpec((BM * D,), lambda i: (i,)),
            pl.BlockSpec((BM, D), lambda i: (i, 0)),
        ],
        out_shape=[
            jax.ShapeDtypeStruct((N * D,), jnp.bfloat16),
            jax.ShapeDtypeStruct((N, D), jnp.float32),
        ],
    )(x, wlT, wrT, b)


def _mean(agg_a_ref, agg_b_ref, deg_a_ref, deg_b_ref, z_ref):
    deg = deg_a_ref[...] + deg_b_ref[...]
    rdeg = 1.0 / jnp.maximum(deg, 1.0)
    agg = agg_a_ref[...].astype(jnp.float32) + agg_b_ref[...].astype(jnp.float32)
    return agg * rdeg + z_ref[...]


def _tc_mid_body(agg_a_ref, agg_b_ref, deg_a_ref, deg_b_ref, z_ref,
                 wl_ref, wr_ref, b_ref, y_ref, zo_ref):
    h = jnp.maximum(_mean(agg_a_ref, agg_b_ref, deg_a_ref, deg_b_ref, z_ref), 0.0)
    y = jnp.dot(h, wl_ref[...], preferred_element_type=jnp.float32)
    y_ref[...] = y.astype(jnp.bfloat16).reshape(BM * D)
    zo_ref[...] = (
        jnp.dot(h, wr_ref[...], preferred_element_type=jnp.float32) + b_ref[...]
    )


def _tc_mid(agg_a, agg_b, deg_a, deg_b, z, wlT, wrT, b):
    return pl.pallas_call(
        _tc_mid_body,
        grid=(N // BM,),
        in_specs=[
            pl.BlockSpec((BM, D), lambda i: (i, 0)),
            pl.BlockSpec((BM, D), lambda i: (i, 0)),
            pl.BlockSpec((BM, 1), lambda i: (i, 0)),
            pl.BlockSpec((BM, 1), lambda i: (i, 0)),
            pl.BlockSpec((BM, D), lambda i: (i, 0)),
            pl.BlockSpec((D, D), lambda i: (0, 0)),
            pl.BlockSpec((D, D), lambda i: (0, 0)),
            pl.BlockSpec((1, D), lambda i: (0, 0)),
        ],
        out_specs=[
            pl.BlockSpec((BM * D,), lambda i: (i,)),
            pl.BlockSpec((BM, D), lambda i: (i, 0)),
        ],
        out_shape=[
            jax.ShapeDtypeStruct((N * D,), jnp.bfloat16),
            jax.ShapeDtypeStruct((N, D), jnp.float32),
        ],
    )(agg_a, agg_b, deg_a, deg_b, z, wlT, wrT, b)


def _tc_fin_body(agg_a_ref, agg_b_ref, deg_a_ref, deg_b_ref, z_ref, o_ref):
    o_ref[...] = _mean(agg_a_ref, agg_b_ref, deg_a_ref, deg_b_ref, z_ref)


def _tc_fin(agg_a, agg_b, deg_a, deg_b, z):
    return pl.pallas_call(
        _tc_fin_body,
        grid=(N // BM,),
        in_specs=[
            pl.BlockSpec((BM, D), lambda i: (i, 0)),
            pl.BlockSpec((BM, D), lambda i: (i, 0)),
            pl.BlockSpec((BM, 1), lambda i: (i, 0)),
            pl.BlockSpec((BM, 1), lambda i: (i, 0)),
            pl.BlockSpec((BM, D), lambda i: (i, 0)),
        ],
        out_specs=pl.BlockSpec((BM, D), lambda i: (i, 0)),
        out_shape=jax.ShapeDtypeStruct((N, D), jnp.float32),
    )(agg_a, agg_b, deg_a, deg_b, z)


def kernel(x, edge_index, W0l, b0l, W0r, W1l, b1l, W1r, W2l, b2l, W2r):
    src = edge_index[0].astype(jnp.int32)
    dst = edge_index[1].astype(jnp.int32)
    eidx = jnp.stack([src.reshape(ECH, CHUNK), dst.reshape(ECH, CHUNK)], axis=1)
    zeros = jnp.zeros((RPT, D), jnp.bfloat16)
    zdeg = jnp.zeros((DR, 16), jnp.float32)
    iota = jnp.arange(DR, dtype=jnp.int32)

    deg_parts = _sc_deg(dst, zdeg, iota)
    dp = deg_parts.reshape(NC, NP)
    deg_a = dp[0].reshape(NP, 1)
    deg_b = dp[1].reshape(NP, 1)

    y0, z0 = _tc_pre(x, W0l.T, W0r.T, b0l.reshape(1, D))
    a0a, a0b = _sc_agg(y0.reshape(N, D), eidx, zeros)

    y1, z1 = _tc_mid(a0a, a0b, deg_a, deg_b, z0, W1l.T, W1r.T, b1l.reshape(1, D))
    a1a, a1b = _sc_agg(y1.reshape(N, D), eidx, zeros)

    y2, z2 = _tc_mid(a1a, a1b, deg_a, deg_b, z1, W2l.T, W2r.T, b2l.reshape(1, D))
    a2a, a2b = _sc_agg(y2.reshape(N, D), eidx, zeros)

    return _tc_fin(a2a, a2b, deg_a, deg_b, z2)


# speedup vs baseline: 14.6341x; 1.0090x over previous
import jax
import jax.numpy as jnp
from jax import lax
from jax.experimental import pallas as pl
from jax.experimental.pallas import tpu as pltpu
from jax.experimental.pallas import tpu_sc as plsc

N = 10000
E = 320000
D = 128

NC = 2
NS = 16
NW = NC * NS
EPT = E // NW
CHUNK = 125
NCHUNK = EPT // CHUNK
ECH = E // CHUNK
NP = 10240
RPT = NP // NS
DR = NP // 16
DRT = DR // NS


def _sc_agg_make():
    mesh = plsc.VectorSubcoreMesh(core_axis_name="c", subcore_axis_name="s")
    scratch = (
        [pltpu.VMEM((CHUNK, D), jnp.bfloat16) for _ in range(8)]
        + [pltpu.VMEM((2, NCHUNK, CHUNK), jnp.int32)]
        + [pltpu.VMEM_SHARED((NP, D), jnp.bfloat16)]
        + [pltpu.SemaphoreType.DMA for _ in range(16)]
    )

    def body(y_hbm, ei_hbm, zeros_hbm, agg_a_out, agg_b_out, *rest):
        rb = rest[0:8]
        eidx_v = rest[8]
        acc_sh = rest[9]
        sg = rest[10:18]
        ss = rest[18:26]
        cid = lax.axis_index("c")
        sid = lax.axis_index("s")
        wid = sid * NC + cid

        y2 = y_hbm
        ei3 = ei_hbm
        zeros2 = zeros_hbm
        agg_a2 = agg_a_out
        agg_b2 = agg_b_out

        pltpu.sync_copy(ei3.at[0, pl.ds(wid * NCHUNK, NCHUNK)], eidx_v.at[0])
        pltpu.sync_copy(ei3.at[1, pl.ds(wid * NCHUNK, NCHUNK)], eidx_v.at[1])
        pltpu.sync_copy(zeros2, acc_sh.at[pl.ds(sid * RPT, RPT)])
        plsc.subcore_barrier()

        def gdesc(c, j):
            return pltpu.make_async_copy(
                y2.at[eidx_v.at[0, c]], rb[j % 8], sg[j % 8]
            )

        def sdesc(c, j):
            return pltpu.make_async_copy(
                rb[j % 8], acc_sh.at[eidx_v.at[1, c]], ss[j % 8]
            )

        def step(c, j):
            @pl.when(jnp.logical_and(c >= 8, c <= NCHUNK + 7))
            def _():
                sdesc(c - 8, j).wait()

            @pl.when(c <= NCHUNK - 1)
            def _():
                gdesc(c, j).start()

            @pl.when(jnp.logical_and(c >= 1, c <= NCHUNK))
            def _():
                gdesc(c - 1, j - 1).wait()
                sdesc(c - 1, j - 1).start(add=True)

        def kbody(k, _):
            c0 = 8 * k
            for j in range(8):
                step(c0 + j, j)
            return 0

        lax.fori_loop(0, (NCHUNK + 15) // 8, kbody, 0)
        plsc.subcore_barrier()

        @pl.when(cid == 0)
        def _():
            pltpu.sync_copy(
                acc_sh.at[pl.ds(sid * RPT, RPT)],
                agg_a2.at[pl.ds(sid * RPT, RPT)],
            )

        @pl.when(cid == 1)
        def _():
            pltpu.sync_copy(
                acc_sh.at[pl.ds(sid * RPT, RPT)],
                agg_b2.at[pl.ds(sid * RPT, RPT)],
            )

    return pl.kernel(
        body,
        out_type=[
            jax.ShapeDtypeStruct((NP, D), jnp.bfloat16),
            jax.ShapeDtypeStruct((NP, D), jnp.bfloat16),
        ],
        mesh=mesh,
        scratch_types=scratch,
        compiler_params=pltpu.CompilerParams(
            needs_layout_passes=False, use_tc_tiling_on_sc=False
        ),
    )


def _sc_deg_make():
    mesh = plsc.VectorSubcoreMesh(core_axis_name="c", subcore_axis_name="s")
    scratch = [
        pltpu.VMEM((EPT,), jnp.int32),
        pltpu.VMEM((DR, 16), jnp.float32),
        pltpu.VMEM((DR,), jnp.int32),
        pltpu.VMEM_SHARED((DR, 16), jnp.float32),
    ]

    def body(ei_hbm, zdeg_hbm, iota_hbm, deg_out, dst_v, deg_v, id_v, deg_sh):
        cid = lax.axis_index("c")
        sid = lax.axis_index("s")
        wid = sid * NC + cid

        dst_hbm = ei_hbm.at[1]
        pltpu.sync_copy(dst_hbm.at[pl.ds(wid * EPT, EPT)], dst_v)
        pltpu.sync_copy(zdeg_hbm, deg_v)
        pltpu.sync_copy(iota_hbm, id_v)
        pltpu.sync_copy(
            zdeg_hbm.at[pl.ds(0, DRT)], deg_sh.at[pl.ds(sid * DRT, DRT)]
        )
        plsc.subcore_barrier()

        ones16 = jnp.full((16,), 1.0, jnp.float32)

        def dbody(i, _):
            dvals = dst_v[pl.ds(i * 16, 16)]
            plsc.addupdate_scatter(deg_v, [dvals >> 4, dvals & 15], ones16)
            return 0

        lax.fori_loop(0, EPT // 16, dbody, 0)

        pltpu.sync_copy(deg_v, deg_sh.at[id_v], add=True)
        plsc.subcore_barrier()
        pltpu.sync_copy(
            deg_sh.at[pl.ds(sid * DRT, DRT)],
            deg_out.at[pl.ds(cid * DR + sid * DRT, DRT)],
        )

    return pl.kernel(
        body,
        out_type=jax.ShapeDtypeStruct((NC * DR, 16), jnp.float32),
        mesh=mesh,
        scratch_types=scratch,
        compiler_params=pltpu.CompilerParams(
            needs_layout_passes=False, use_tc_tiling_on_sc=False
        ),
    )


_sc_agg = _sc_agg_make()
_sc_deg = _sc_deg_make()


BM = 2000


def _tc_pre_body(x_ref, wl_ref, wr_ref, b_ref, y_ref, z_ref):
    xb = x_ref[...]
    y = jnp.dot(xb, wl_ref[...], preferred_element_type=jnp.float32)
    y_ref[...] = y.astype(jnp.bfloat16).reshape(BM * D)
    z_ref[...] = (
        jnp.dot(xb, wr_ref[...], preferred_element_type=jnp.float32) + b_ref[...]
    )


def _tc_pre(x, wlT, wrT, b):
    return pl.pallas_call(
        _tc_pre_body,
        grid=(N // BM,),
        in_specs=[
            pl.BlockSpec((BM, D), lambda i: (i, 0)),
            pl.BlockSpec((D, D), lambda i: (0, 0)),
            pl.BlockSpec((D, D), lambda i: (0, 0)),
            pl.BlockSpec((1, D), lambda i: (0, 0)),
        ],
        out_specs=[
            pl.BlockSpec((BM * D,), lambda i: (i,)),
            pl.BlockSpec((BM, D), lambda i: (i, 0)),
        ],
        out_shape=[
            jax.ShapeDtypeStruct((N * D,), jnp.bfloat16),
            jax.ShapeDtypeStruct((N, D), jnp.float32),
        ],
    )(x, wlT, wrT, b)


def _mean(agg_a_ref, agg_b_ref, deg_a_ref, deg_b_ref, z_ref):
    deg = deg_a_ref[...] + deg_b_ref[...]
    rdeg = 1.0 / jnp.maximum(deg, 1.0)
    agg = agg_a_ref[...].astype(jnp.float32) + agg_b_ref[...].astype(jnp.float32)
    return agg * rdeg + z_ref[...]


def _mean_flat(agg_a_ref, agg_b_ref, deg_a_ref, deg_b_ref, z_ref):
    deg = deg_a_ref[...] + deg_b_ref[...]
    rdeg = 1.0 / jnp.maximum(deg, 1.0)
    agg = (
        agg_a_ref[...].reshape(BM, D).astype(jnp.float32)
        + agg_b_ref[...].reshape(BM, D).astype(jnp.float32)
    )
    return agg * rdeg + z_ref[...]


def _tc_mid_body(agg_a_ref, agg_b_ref, deg_a_ref, deg_b_ref, z_ref,
                 wl_ref, wr_ref, b_ref, y_ref, zo_ref):
    h = jnp.maximum(
        _mean_flat(agg_a_ref, agg_b_ref, deg_a_ref, deg_b_ref, z_ref), 0.0
    )
    y = jnp.dot(h, wl_ref[...], preferred_element_type=jnp.float32)
    y_ref[...] = y.astype(jnp.bfloat16).reshape(BM * D)
    zo_ref[...] = (
        jnp.dot(h, wr_ref[...], preferred_element_type=jnp.float32) + b_ref[...]
    )


def _tc_mid(agg_a, agg_b, deg_a, deg_b, z, wlT, wrT, b):
    return pl.pallas_call(
        _tc_mid_body,
        grid=(N // BM,),
        in_specs=[
            pl.BlockSpec((BM * D,), lambda i: (i,)),
            pl.BlockSpec((BM * D,), lambda i: (i,)),
            pl.BlockSpec((BM, 1), lambda i: (i, 0)),
            pl.BlockSpec((BM, 1), lambda i: (i, 0)),
            pl.BlockSpec((BM, D), lambda i: (i, 0)),
            pl.BlockSpec((D, D), lambda i: (0, 0)),
            pl.BlockSpec((D, D), lambda i: (0, 0)),
            pl.BlockSpec((1, D), lambda i: (0, 0)),
        ],
        out_specs=[
            pl.BlockSpec((BM * D,), lambda i: (i,)),
            pl.BlockSpec((BM, D), lambda i: (i, 0)),
        ],
        out_shape=[
            jax.ShapeDtypeStruct((N * D,), jnp.bfloat16),
            jax.ShapeDtypeStruct((N, D), jnp.float32),
        ],
    )(agg_a, agg_b, deg_a, deg_b, z, wlT, wrT, b)


def _tc_fin_body(agg_a_ref, agg_b_ref, deg_a_ref, deg_b_ref, z_ref, o_ref):
    o_ref[...] = _mean_flat(agg_a_ref, agg_b_ref, deg_a_ref, deg_b_ref, z_ref)


def _tc_fin(agg_a, agg_b, deg_a, deg_b, z):
    return pl.pallas_call(
        _tc_fin_body,
        grid=(N // BM,),
        in_specs=[
            pl.BlockSpec((BM * D,), lambda i: (i,)),
            pl.BlockSpec((BM * D,), lambda i: (i,)),
            pl.BlockSpec((BM, 1), lambda i: (i, 0)),
            pl.BlockSpec((BM, 1), lambda i: (i, 0)),
            pl.BlockSpec((BM, D), lambda i: (i, 0)),
        ],
        out_specs=pl.BlockSpec((BM, D), lambda i: (i, 0)),
        out_shape=jax.ShapeDtypeStruct((N, D), jnp.float32),
    )(agg_a, agg_b, deg_a, deg_b, z)


def kernel(x, edge_index, W0l, b0l, W0r, W1l, b1l, W1r, W2l, b2l, W2r):
    ei = edge_index.astype(jnp.int32)
    ei3 = ei.reshape(2, ECH, CHUNK)
    zeros = jnp.zeros((RPT, D), jnp.bfloat16)
    zdeg = jnp.zeros((DR, 16), jnp.float32)
    iota = jnp.arange(DR, dtype=jnp.int32)

    deg_parts = _sc_deg(ei, zdeg, iota)
    dp = deg_parts.reshape(NC, NP)
    deg_a = dp[0].reshape(NP, 1)
    deg_b = dp[1].reshape(NP, 1)

    y0, z0 = _tc_pre(x, W0l.T, W0r.T, b0l.reshape(1, D))
    a0a, a0b = _sc_agg(y0.reshape(N, D), ei3, zeros)

    y1, z1 = _tc_mid(a0a.reshape(NP * D), a0b.reshape(NP * D), deg_a, deg_b,
                     z0, W1l.T, W1r.T, b1l.reshape(1, D))
    a1a, a1b = _sc_agg(y1.reshape(N, D), ei3, zeros)

    y2, z2 = _tc_mid(a1a.reshape(NP * D), a1b.reshape(NP * D), deg_a, deg_b,
                     z1, W2l.T, W2r.T, b2l.reshape(1, D))
    a2a, a2b = _sc_agg(y2.reshape(N, D), ei3, zeros)

    return _tc_fin(a2a.reshape(NP * D), a2b.reshape(NP * D), deg_a, deg_b, z2)
